# baseline (device time: 21702 ns/iter reference)
import jax
import jax.numpy as jnp
from jax import lax
from jax.experimental import pallas as pl
from jax.experimental.pallas import tpu as pltpu

N_DEV = 4
N_TOK = 512
D_IN = 256
D_OUT = 512
N_EXP = 8
CAP = 51
CHUNK = N_TOK // N_DEV


def kernel(x, router_W, route_idx, expert_W):
    def body(x_ref, rw_ref, idx_ref, w_ref, out_ref,
             xm_ref, send_buf, recv_buf, send_sems, recv_sems):
        p = lax.axis_index("i")
        left = lax.rem(p + N_DEV - 1, N_DEV)
        right = lax.rem(p + 1, N_DEV)

        bar = pltpu.get_barrier_semaphore()
        pl.semaphore_signal(bar, inc=1, device_id=(left,),
                            device_id_type=pl.DeviceIdType.MESH)
        pl.semaphore_signal(bar, inc=1, device_id=(right,),
                            device_id_type=pl.DeviceIdType.MESH)
        pl.semaphore_wait(bar, 2)

        e = idx_ref[:, :]
        oh = jnp.where(
            e == lax.broadcasted_iota(jnp.int32, (N_TOK, N_EXP), 1),
            1.0, 0.0).astype(jnp.float32)
        row = lax.broadcasted_iota(jnp.int32, (N_TOK, N_TOK), 0)
        col = lax.broadcasted_iota(jnp.int32, (N_TOK, N_TOK), 1)
        tri = jnp.where(col <= row, 1.0, 0.0).astype(jnp.float32)
        counts = jnp.dot(tri, oh, preferred_element_type=jnp.float32)
        myc = jnp.sum(oh * counts, axis=1, keepdims=True)
        keep = myc <= float(CAP)

        for le in range(2):
            ge = 2 * p + le
            m = jnp.where(keep & (e == ge), 1.0, 0.0).astype(jnp.float32)
            xm_ref[le, :, :] = x_ref[:, :] * m

        def contrib(c):
            start = c * CHUNK
            a0 = xm_ref[0, pl.ds(start, CHUNK), :]
            a1 = xm_ref[1, pl.ds(start, CHUNK), :]
            return (jnp.dot(a0, w_ref[0], preferred_element_type=jnp.float32)
                    + jnp.dot(a1, w_ref[1], preferred_element_type=jnp.float32))

        acc = contrib(lax.rem(p + N_DEV - 1, N_DEV))
        for h in range(N_DEV - 1):
            send_buf[h, :, :] = acc
            rdma = pltpu.make_async_remote_copy(
                src_ref=send_buf.at[h],
                dst_ref=recv_buf.at[h],
                send_sem=send_sems.at[h],
                recv_sem=recv_sems.at[h],
                device_id=(right,),
                device_id_type=pl.DeviceIdType.MESH,
            )
            rdma.start()
            nxt = contrib(lax.rem(p + 2 * N_DEV - 2 - h, N_DEV))
            rdma.wait()
            acc = recv_buf[h, :, :] + nxt
        out_ref[:, :] = acc

    return pl.pallas_call(
        body,
        out_shape=jax.ShapeDtypeStruct((CHUNK, D_OUT), jnp.float32),
        in_specs=[
            pl.BlockSpec(memory_space=pltpu.MemorySpace.VMEM),
            pl.BlockSpec(memory_space=pltpu.MemorySpace.VMEM),
            pl.BlockSpec(memory_space=pltpu.MemorySpace.VMEM),
            pl.BlockSpec(memory_space=pltpu.MemorySpace.VMEM),
        ],
        out_specs=pl.BlockSpec(memory_space=pltpu.MemorySpace.VMEM),
        scratch_shapes=[
            pltpu.VMEM((2, N_TOK, D_IN), jnp.float32),
            pltpu.VMEM((N_DEV - 1, CHUNK, D_OUT), jnp.float32),
            pltpu.VMEM((N_DEV - 1, CHUNK, D_OUT), jnp.float32),
            pltpu.SemaphoreType.DMA((N_DEV - 1,)),
            pltpu.SemaphoreType.DMA((N_DEV - 1,)),
        ],
        compiler_params=pltpu.CompilerParams(collective_id=0),
    )(x, router_W, route_idx, expert_W)


# device time: 16684 ns/iter; 1.3008x vs baseline; 1.3008x over previous
import functools

import jax
import jax.numpy as jnp
from jax import lax
from jax.experimental import pallas as pl
from jax.experimental.pallas import tpu as pltpu

N_DEV = 4
N_TOK = 512
D_IN = 256
D_OUT = 512
N_EXP = 8
CAP = 51
CHUNK = N_TOK // N_DEV


def kernel(x, router_W, route_idx, expert_W):
    def body(x_ref, rw_ref, idx_ref, w_ref, out_ref,
             xm_ref, send_buf, recv_buf, send_sems, recv_sems):
        p = lax.axis_index("i")
        peers = [lax.rem(p + o, N_DEV) for o in (1, 2, 3)]

        bar = pltpu.get_barrier_semaphore()
        for d in peers:
            pl.semaphore_signal(bar, inc=1, device_id=(d,),
                                device_id_type=pl.DeviceIdType.MESH)

        e = idx_ref[:, :]
        oh = jnp.where(
            e == lax.broadcasted_iota(jnp.int32, (N_TOK, N_EXP), 1),
            1.0, 0.0).astype(jnp.float32)
        row = lax.broadcasted_iota(jnp.int32, (N_TOK, N_TOK), 0)
        col = lax.broadcasted_iota(jnp.int32, (N_TOK, N_TOK), 1)
        tri = jnp.where(col <= row, 1.0, 0.0).astype(jnp.float32)
        counts = jnp.dot(tri, oh, preferred_element_type=jnp.float32)
        myc = jnp.sum(oh * counts, axis=1, keepdims=True)
        keep = myc <= float(CAP)

        for le in range(2):
            ge = 2 * p + le
            m = jnp.where(keep & (e == ge), 1.0, 0.0).astype(jnp.float32)
            xm_ref[le, :, :] = x_ref[:, :] * m

        def contrib(c):
            start = c * CHUNK
            a0 = xm_ref[0, pl.ds(start, CHUNK), :]
            a1 = xm_ref[1, pl.ds(start, CHUNK), :]
            return (jnp.dot(a0, w_ref[0], preferred_element_type=jnp.float32)
                    + jnp.dot(a1, w_ref[1], preferred_element_type=jnp.float32))

        pl.semaphore_wait(bar, N_DEV - 1)

        rdmas = []
        for j, o in enumerate((2, 1, 3)):
            dst = lax.rem(p + o, N_DEV)
            send_buf[o - 1, :, :] = contrib(dst)
            rdma = pltpu.make_async_remote_copy(
                src_ref=send_buf.at[o - 1],
                dst_ref=recv_buf.at[o - 1],
                send_sem=send_sems.at[o - 1],
                recv_sem=recv_sems.at[o - 1],
                device_id=(dst,),
                device_id_type=pl.DeviceIdType.MESH,
            )
            rdma.start()
            rdmas.append(rdma)

        acc = contrib(p)
        for j, o in enumerate((2, 1, 3)):
            rdmas[j].wait_recv()
            acc = acc + recv_buf[o - 1, :, :]
        out_ref[:, :] = acc
        for rdma in rdmas:
            rdma.wait_send()

        @functools.partial(pl.run_scoped, exit_bar=pltpu.SemaphoreType.REGULAR)
        def _(exit_bar):
            for d in peers:
                pl.semaphore_signal(exit_bar, inc=1, device_id=(d,),
                                    device_id_type=pl.DeviceIdType.MESH)
            pl.semaphore_wait(exit_bar, N_DEV - 1)

    return pl.pallas_call(
        body,
        out_shape=jax.ShapeDtypeStruct((CHUNK, D_OUT), jnp.float32),
        in_specs=[
            pl.BlockSpec(memory_space=pltpu.MemorySpace.VMEM),
            pl.BlockSpec(memory_space=pltpu.MemorySpace.VMEM),
            pl.BlockSpec(memory_space=pltpu.MemorySpace.VMEM),
            pl.BlockSpec(memory_space=pltpu.MemorySpace.VMEM),
        ],
        out_specs=pl.BlockSpec(memory_space=pltpu.MemorySpace.VMEM),
        scratch_shapes=[
            pltpu.VMEM((2, N_TOK, D_IN), jnp.float32),
            pltpu.VMEM((N_DEV - 1, CHUNK, D_OUT), jnp.float32),
            pltpu.VMEM((N_DEV - 1, CHUNK, D_OUT), jnp.float32),
            pltpu.SemaphoreType.DMA((N_DEV - 1,)),
            pltpu.SemaphoreType.DMA((N_DEV - 1,)),
        ],
        compiler_params=pltpu.CompilerParams(collective_id=0),
    )(x, router_W, route_idx, expert_W)
